# Initial kernel scaffold; baseline (speedup 1.0000x reference)
import jax
import jax.numpy as jnp
from jax.experimental import pallas as pl


def kernel(x, emb, ln_scale, ln_bias, Wq, Wk, Wv, Wo, Wrq, Wwk, Wwv, A, fn_scale, fn_bias, Wout, bout):
    raise NotImplementedError("write your pallas kernel here")


# speedup vs baseline: 10.7506x; 10.7506x over previous
import jax
import jax.numpy as jnp
import numpy as np
from jax.experimental import pallas as pl
from jax.experimental.pallas import tpu as pltpu

_B, _S, _D, _V = 1, 2048, 768, 1000
_NL, _K, _KR, _CHUNK = 2, 128, 16, 256
_H = 8
_DH = _D // _H
_NC = _S // _CHUNK
_SQD = float(np.sqrt(_DH))
_NEG = -1e30
_HI = jax.lax.Precision.HIGHEST


def _ln(x, scale, bias):
    mu = jnp.mean(x, axis=-1, keepdims=True)
    var = jnp.var(x, axis=-1, keepdims=True)
    return (x - mu) / jnp.sqrt(var + 1e-5) * scale + bias


def _topk_softmax(s):
    col = jax.lax.broadcasted_iota(jnp.int32, s.shape, 1)
    m1 = jnp.max(s, axis=-1, keepdims=True)
    tmp = s
    sel = jnp.zeros(s.shape, jnp.bool_)
    for _ in range(_KR):
        m = jnp.max(tmp, axis=-1, keepdims=True)
        is_max = tmp == m
        idx = jnp.min(jnp.where(is_max, col, _K), axis=-1, keepdims=True)
        pick = is_max & (col == idx)
        sel = sel | pick
        tmp = jnp.where(pick, _NEG, tmp)
    e = jnp.where(sel, jnp.exp(s - m1), 0.0)
    return e, jnp.sum(e, axis=-1, keepdims=True)


def _split3(x):
    h = x.astype(jnp.bfloat16).astype(jnp.float32)
    r = x - h
    m = r.astype(jnp.bfloat16).astype(jnp.float32)
    return h, m, r - m


def _exact_dot_t(a, b):
    ah, am, al = _split3(a)
    bh, bm, bl = _split3(b)
    dn = (((0,), (0,)), ((), ()))
    dg = lambda u, v: jax.lax.dot_general(
        u, v, dn, preferred_element_type=jnp.float32)
    return ((dg(ah, bh) + (dg(ah, bm) + dg(am, bh)))
            + ((dg(ah, bl) + dg(al, bh)) + dg(am, bm)))


def _embed_kernel(x_ref, emb_ref, o_ref):
    ids = x_ref[...]
    oh = (ids == jax.lax.broadcasted_iota(jnp.int32, (_CHUNK, _V), 1))
    o_ref[...] = jnp.dot(oh.astype(jnp.float32), emb_ref[...],
                         preferred_element_type=jnp.float32, precision=_HI)


def _layer_kernel(cn_ref, res_ref, mk_ref, mv_ref, wq_ref, wk_ref, wv_ref,
                  wo_ref, wrq_ref, wwk_ref, wwv_ref, a_ref,
                  out_ref, mko_ref, mvo_ref, comb):
    cn = cn_ref[...]
    q = jnp.dot(cn, wq_ref[...], preferred_element_type=jnp.float32)
    k = jnp.dot(cn, wk_ref[...], preferred_element_type=jnp.float32)
    v = jnp.dot(cn, wv_ref[...], preferred_element_type=jnp.float32)
    rq = jnp.dot(cn, wrq_ref[...], preferred_element_type=jnp.float32)
    wkp = jnp.dot(cn, wwk_ref[...], preferred_element_type=jnp.float32)
    wvp = jnp.dot(cn, wwv_ref[...], preferred_element_type=jnp.float32)
    ri = jax.lax.broadcasted_iota(jnp.int32, (_CHUNK, _CHUNK), 0)
    ci = jax.lax.broadcasted_iota(jnp.int32, (_CHUNK, _CHUNK), 1)
    causal = ri >= ci
    for h in range(_H):
        sl = slice(h * _DH, (h + 1) * _DH)
        qh, kh, vh = q[:, sl], k[:, sl], v[:, sl]
        s = jax.lax.dot_general(
            qh, kh, (((1,), (1,)), ((), ())),
            preferred_element_type=jnp.float32) / _SQD
        s = jnp.where(causal, s, _NEG)
        sm = jnp.max(s, axis=-1, keepdims=True)
        se = jnp.exp(s - sm)
        local = jnp.dot(se, vh, preferred_element_type=jnp.float32) \
            / jnp.sum(se, axis=-1, keepdims=True)
        rs = jax.lax.dot_general(
            rq[:, sl], mk_ref[h], (((1,), (1,)), ((), ())),
            preferred_element_type=jnp.float32) / _SQD
        re, rden = _topk_softmax(rs)
        ro = jnp.dot(re / rden, mv_ref[h], preferred_element_type=jnp.float32)
        comb[:, sl] = local + ro
        wkh = wkp[:, sl]
        ws = jax.lax.dot_general(
            wkh, a_ref[h], (((1,), (1,)), ((), ())),
            preferred_element_type=jnp.float32) / _SQD
        we, wden = _topk_softmax(ws)
        ww = we / wden
        mko_ref[h] = mk_ref[h] + _exact_dot_t(ww, wkh)
        mvo_ref[h] = mv_ref[h] + _exact_dot_t(ww, wvp[:, sl])
    out = jnp.dot(comb[...], wo_ref[...], preferred_element_type=jnp.float32)
    out_ref[...] = res_ref[...] + out


def _logits_kernel(y_ref, wout_ref, bout_ref, o_ref):
    o_ref[...] = jnp.dot(y_ref[...], wout_ref[...],
                         preferred_element_type=jnp.float32) + bout_ref[...]


def _embed(x, emb):
    return pl.pallas_call(
        _embed_kernel,
        grid=(_NC,),
        in_specs=[
            pl.BlockSpec((_CHUNK, 1), lambda c: (c, 0)),
            pl.BlockSpec((_V, _D), lambda c: (0, 0)),
        ],
        out_specs=pl.BlockSpec((_CHUNK, _D), lambda c: (c, 0)),
        out_shape=jax.ShapeDtypeStruct((_S, _D), jnp.float32),
    )(x.reshape(_S, 1).astype(jnp.int32), emb)


def _layer(cn, res, mk, mv, wq, wk, wv, wo, wrq, wwk, wwv, a):
    mem_t = jax.ShapeDtypeStruct((_H, _K, _DH), jnp.float32)
    return pl.pallas_call(
        _layer_kernel,
        out_shape=(jax.ShapeDtypeStruct((_CHUNK, _D), jnp.float32),
                   mem_t, mem_t),
        scratch_shapes=[pltpu.VMEM((_CHUNK, _D), jnp.float32)],
        compiler_params=pltpu.CompilerParams(
            vmem_limit_bytes=100 * 1024 * 1024),
    )(cn, res, mk, mv, wq, wk, wv, wo, wrq, wwk, wwv, a)


def _logits(y, wout, bout):
    return pl.pallas_call(
        _logits_kernel,
        out_shape=jax.ShapeDtypeStruct((_S, _V), jnp.float32),
    )(y, wout, bout.reshape(1, _V))


def kernel(x, emb, ln_scale, ln_bias, Wq, Wk, Wv, Wo, Wrq, Wwk, Wwv, A,
           fn_scale, fn_bias, Wout, bout):
    hidden = _embed(x, emb).reshape(1, _S, _D)
    mk = [jnp.zeros((_H, _K, _DH), jnp.float32) for _ in range(_NL)]
    mv = [jnp.zeros((_H, _K, _DH), jnp.float32) for _ in range(_NL)]
    outs = []
    for c in range(_NC):
        chunk = hidden[:, c * _CHUNK:(c + 1) * _CHUNK, :]
        for li in range(_NL):
            cn = _ln(chunk, ln_scale[li], ln_bias[li])
            nxt, mk[li], mv[li] = _layer(
                cn[0], chunk[0], mk[li], mv[li], Wq[li], Wk[li], Wv[li],
                Wo[li], Wrq[li], Wwk[li], Wwv[li], A[li])
            chunk = nxt.reshape(1, _CHUNK, _D)
        outs.append(chunk)
    out = jnp.concatenate(outs, axis=1)
    y = _ln(out, fn_scale, fn_bias)
    return _logits(y[0], Wout, bout).reshape(_B, _S, _V)
